# Initial kernel scaffold; baseline (speedup 1.0000x reference)
import jax
import jax.numpy as jnp
from jax.experimental import pallas as pl


def kernel(x, edge_index, W_proj, b_proj, W_gcn, b_gcn, W_out, b_out):
    raise NotImplementedError("write your pallas kernel here")


# speedup vs baseline: 100.9065x; 100.9065x over previous
import functools

import jax
import jax.numpy as jnp
from jax import lax
from jax.experimental import pallas as pl
from jax.experimental.pallas import tpu as pltpu
from jax.experimental.pallas import tpu_sc as plsc

N = 10000
E = 320000
NC = 2
NS = 16
L = 16
NW = NC * NS
NP = 10240
EP = 327680
EPT = EP // NW
BB = 128
NB = EPT // BB
NSL = NP // NS


def _sc_mesh():
    return plsc.VectorSubcoreMesh(core_axis_name="c", subcore_axis_name="s")


_SC_PARAMS = pltpu.CompilerParams(needs_layout_passes=False)


def _deg_body(dst2d_hbm, out_hbm, didx_v, ones_v, zero_v, acc_sh):
    cid = lax.axis_index("c")
    sid = lax.axis_index("s")
    wid = cid * NS + sid
    pltpu.sync_copy(dst2d_hbm.at[pl.ds(wid * NB, NB)], didx_v)
    for i in range(BB // L):
        ones_v[pl.ds(i * L, L)] = jnp.ones((L,), jnp.float32)
    for i in range(NSL // L):
        zero_v[pl.ds(i * L, L)] = jnp.zeros((L,), jnp.float32)
    pltpu.sync_copy(zero_v, acc_sh.at[pl.ds(sid * NSL, NSL)])
    plsc.subcore_barrier()

    def batch(j, carry):
        pltpu.sync_copy(ones_v, acc_sh.at[didx_v.at[j]], add=True)
        return carry

    lax.fori_loop(0, NB, batch, 0)
    plsc.subcore_barrier()
    pltpu.sync_copy(acc_sh.at[pl.ds(sid * NSL, NSL)],
                    out_hbm.at[cid, pl.ds(sid * NSL, NSL)])


def _degree_partials(dst2d):
    return pl.kernel(
        _deg_body,
        out_type=jax.ShapeDtypeStruct((NC, NP), jnp.float32),
        mesh=_sc_mesh(),
        compiler_params=_SC_PARAMS,
        scratch_types=[
            pltpu.VMEM((NB, BB), jnp.int32),
            pltpu.VMEM((BB,), jnp.float32),
            pltpu.VMEM((NSL,), jnp.float32),
            pltpu.VMEM_SHARED((NP,), jnp.float32),
        ],
    )(dst2d)


def _msg_body(src_hbm, dst2d_hbm, t_hbm, out_hbm,
              sidx_v, didx_v, vals_v, t_v, zero_v, acc_sh):
    cid = lax.axis_index("c")
    sid = lax.axis_index("s")
    wid = cid * NS + sid
    pltpu.sync_copy(src_hbm.at[pl.ds(wid * EPT, EPT)], sidx_v)
    pltpu.sync_copy(dst2d_hbm.at[pl.ds(wid * NB, NB)], didx_v)
    pltpu.sync_copy(t_hbm, t_v)
    for i in range(NSL // L):
        zero_v[pl.ds(i * L, L)] = jnp.zeros((L,), jnp.float32)
    pltpu.sync_copy(zero_v, acc_sh.at[pl.ds(sid * NSL, NSL)])
    plsc.subcore_barrier()

    def batch(j, carry):
        for k in range(BB // L):
            si = sidx_v[pl.ds(j * BB + k * L, L)]
            vals_v[pl.ds(j * BB + k * L, L)] = plsc.load_gather(t_v, [si])
        pltpu.sync_copy(vals_v.at[pl.ds(j * BB, BB)],
                        acc_sh.at[didx_v.at[j]], add=True)
        return carry

    lax.fori_loop(0, NB, batch, 0)
    plsc.subcore_barrier()
    pltpu.sync_copy(acc_sh.at[pl.ds(sid * NSL, NSL)],
                    out_hbm.at[cid, pl.ds(sid * NSL, NSL)])


def _message_partials(src_p, dst2d, t):
    return pl.kernel(
        _msg_body,
        out_type=jax.ShapeDtypeStruct((NC, NP), jnp.float32),
        mesh=_sc_mesh(),
        compiler_params=_SC_PARAMS,
        scratch_types=[
            pltpu.VMEM((EPT,), jnp.int32),
            pltpu.VMEM((NB, BB), jnp.int32),
            pltpu.VMEM((EPT,), jnp.float32),
            pltpu.VMEM((NP,), jnp.float32),
            pltpu.VMEM((NSL,), jnp.float32),
            pltpu.VMEM_SHARED((NP,), jnp.float32),
        ],
    )(src_p, dst2d, t)


def _mid_body(x_ref, wp_ref, bp_ref, wg_ref, wo_ref, degp_ref,
              t_ref, dinv_ref):
    u = jnp.dot(wo_ref[...][0, :], wg_ref[...])
    h = lax.dot_general(x_ref[...], wp_ref[...],
                        (((1,), (1,)), ((), ())),
                        preferred_element_type=jnp.float32)
    h = jnp.maximum(h + bp_ref[...][None, :], 0.0)
    t_raw = jnp.sum(h * u[None, :], axis=1)
    deg = degp_ref[0, :] + degp_ref[1, :] + 1.0
    dinv = lax.rsqrt(deg)
    t_ref[...] = dinv * t_raw
    dinv_ref[...] = dinv


def _tc_mid(x_pad, W_proj, b_proj, W_gcn, W_out, degp):
    return pl.pallas_call(
        _mid_body,
        out_shape=[
            jax.ShapeDtypeStruct((NP,), jnp.float32),
            jax.ShapeDtypeStruct((NP,), jnp.float32),
        ],
    )(x_pad, W_proj, b_proj, W_gcn, W_out, degp)


def _final_body(dinv_ref, t_ref, sp_ref, wo_ref, bg_ref, bo_ref, out_ref):
    c = jnp.sum(wo_ref[...][0, :] * bg_ref[...]) + jnp.sum(bo_ref[...])
    out_ref[...] = dinv_ref[...] * (sp_ref[0, :] + sp_ref[1, :] + t_ref[...]) + c


def _tc_final(dinv, t, sp, W_out, b_gcn, b_out):
    return pl.pallas_call(
        _final_body,
        out_shape=jax.ShapeDtypeStruct((NP,), jnp.float32),
    )(dinv, t, sp, W_out, b_gcn, b_out)


@jax.jit
def kernel(x, edge_index, W_proj, b_proj, W_gcn, b_gcn, W_out, b_out):
    src = edge_index[0]
    dst = edge_index[1]
    pad = EP - E
    src_p = jnp.concatenate([src, jnp.zeros((pad,), jnp.int32)])
    dst_p = jnp.concatenate([dst, jnp.full((pad,), N + 16, jnp.int32)])
    dst2d = dst_p.reshape(EP // BB, BB)
    x_pad = jnp.pad(x, ((0, NP - N), (0, 0)))

    degp = _degree_partials(dst2d)
    t, dinv = _tc_mid(x_pad, W_proj, b_proj, W_gcn, W_out, degp)
    sp = _message_partials(src_p, dst2d, t)
    out_full = _tc_final(dinv, t, sp, W_out, b_gcn, b_out)
    return out_full[:N, None]


# speedup vs baseline: 103.1780x; 1.0225x over previous
import functools

import jax
import jax.numpy as jnp
from jax import lax
from jax.experimental import pallas as pl
from jax.experimental.pallas import tpu as pltpu
from jax.experimental.pallas import tpu_sc as plsc

N = 10000
E = 320000
NC = 2
NS = 16
L = 16
NW = NC * NS
NP = 10240
EP = 327680
EPT = EP // NW
BB = 128
NB = EPT // BB
NSL = NP // NS


def _sc_mesh():
    return plsc.VectorSubcoreMesh(core_axis_name="c", subcore_axis_name="s")


_SC_PARAMS = pltpu.CompilerParams(needs_layout_passes=False)


_G = 16


def _deg_body(dst2d_hbm, out_hbm, didx_v, ones_v, zero_v, acc_sh, sem):
    cid = lax.axis_index("c")
    sid = lax.axis_index("s")
    wid = cid * NS + sid
    stage = pltpu.async_copy(dst2d_hbm.at[pl.ds(wid * NB, NB)], didx_v, sem)
    for i in range(BB // L):
        ones_v[pl.ds(i * L, L)] = jnp.ones((L,), jnp.float32)
    for i in range(NSL // L):
        zero_v[pl.ds(i * L, L)] = jnp.zeros((L,), jnp.float32)
    pltpu.sync_copy(zero_v, acc_sh.at[pl.ds(sid * NSL, NSL)])
    stage.wait()
    plsc.subcore_barrier()

    def group(g, carry):
        cps = [
            pltpu.async_copy(ones_v, acc_sh.at[didx_v.at[g * _G + jj]],
                             sem, add=True)
            for jj in range(_G)
        ]
        for cp in cps:
            cp.wait()
        return carry

    lax.fori_loop(0, NB // _G, group, 0)
    plsc.subcore_barrier()
    pltpu.sync_copy(acc_sh.at[pl.ds(sid * NSL, NSL)],
                    out_hbm.at[cid, pl.ds(sid * NSL, NSL)])


def _degree_partials(dst2d):
    return pl.kernel(
        _deg_body,
        out_type=jax.ShapeDtypeStruct((NC, NP), jnp.float32),
        mesh=_sc_mesh(),
        compiler_params=_SC_PARAMS,
        scratch_types=[
            pltpu.VMEM((NB, BB), jnp.int32),
            pltpu.VMEM((BB,), jnp.float32),
            pltpu.VMEM((NSL,), jnp.float32),
            pltpu.VMEM_SHARED((NP,), jnp.float32),
            pltpu.SemaphoreType.DMA,
        ],
    )(dst2d)


def _msg_body(src_hbm, dst2d_hbm, t_hbm, out_hbm,
              sidx_v, didx_v, vals_v, t_v, zero_v, acc_sh, sem):
    cid = lax.axis_index("c")
    sid = lax.axis_index("s")
    wid = cid * NS + sid
    st1 = pltpu.async_copy(src_hbm.at[pl.ds(wid * EPT, EPT)], sidx_v, sem)
    st2 = pltpu.async_copy(dst2d_hbm.at[pl.ds(wid * NB, NB)], didx_v, sem)
    st3 = pltpu.async_copy(t_hbm, t_v, sem)
    for i in range(NSL // L):
        zero_v[pl.ds(i * L, L)] = jnp.zeros((L,), jnp.float32)
    pltpu.sync_copy(zero_v, acc_sh.at[pl.ds(sid * NSL, NSL)])
    st1.wait()
    st2.wait()
    st3.wait()
    plsc.subcore_barrier()

    def group(g, carry):
        cps = []
        for jj in range(_G):
            j = g * _G + jj
            for k in range(BB // L):
                off = j * BB + k * L
                si = sidx_v[pl.ds(off, L)]
                vals_v[pl.ds(off, L)] = plsc.load_gather(t_v, [si])
            cps.append(
                pltpu.async_copy(vals_v.at[pl.ds(j * BB, BB)],
                                 acc_sh.at[didx_v.at[j]], sem, add=True))
        for cp in cps:
            cp.wait()
        return carry

    lax.fori_loop(0, NB // _G, group, 0)
    plsc.subcore_barrier()
    pltpu.sync_copy(acc_sh.at[pl.ds(sid * NSL, NSL)],
                    out_hbm.at[cid, pl.ds(sid * NSL, NSL)])


def _message_partials(src_p, dst2d, t):
    return pl.kernel(
        _msg_body,
        out_type=jax.ShapeDtypeStruct((NC, NP), jnp.float32),
        mesh=_sc_mesh(),
        compiler_params=_SC_PARAMS,
        scratch_types=[
            pltpu.VMEM((EPT,), jnp.int32),
            pltpu.VMEM((NB, BB), jnp.int32),
            pltpu.VMEM((EPT,), jnp.float32),
            pltpu.VMEM((NP,), jnp.float32),
            pltpu.VMEM((NSL,), jnp.float32),
            pltpu.VMEM_SHARED((NP,), jnp.float32),
            pltpu.SemaphoreType.DMA,
        ],
    )(src_p, dst2d, t)


def _mid_body(x_ref, wp_ref, bp_ref, wg_ref, wo_ref, degp_ref,
              t_ref, dinv_ref):
    u = jnp.dot(wo_ref[...][0, :], wg_ref[...])
    h = lax.dot_general(x_ref[...], wp_ref[...],
                        (((1,), (1,)), ((), ())),
                        preferred_element_type=jnp.float32)
    h = jnp.maximum(h + bp_ref[...][None, :], 0.0)
    t_raw = jnp.sum(h * u[None, :], axis=1)
    deg = degp_ref[0, :] + degp_ref[1, :] + 1.0
    dinv = lax.rsqrt(deg)
    t_ref[...] = dinv * t_raw
    dinv_ref[...] = dinv


def _tc_mid(x_pad, W_proj, b_proj, W_gcn, W_out, degp):
    return pl.pallas_call(
        _mid_body,
        out_shape=[
            jax.ShapeDtypeStruct((NP,), jnp.float32),
            jax.ShapeDtypeStruct((NP,), jnp.float32),
        ],
    )(x_pad, W_proj, b_proj, W_gcn, W_out, degp)


def _final_body(dinv_ref, t_ref, sp_ref, wo_ref, bg_ref, bo_ref, out_ref):
    c = jnp.sum(wo_ref[...][0, :] * bg_ref[...]) + jnp.sum(bo_ref[...])
    out_ref[...] = dinv_ref[...] * (sp_ref[0, :] + sp_ref[1, :] + t_ref[...]) + c


def _tc_final(dinv, t, sp, W_out, b_gcn, b_out):
    return pl.pallas_call(
        _final_body,
        out_shape=jax.ShapeDtypeStruct((NP,), jnp.float32),
    )(dinv, t, sp, W_out, b_gcn, b_out)


@jax.jit
def kernel(x, edge_index, W_proj, b_proj, W_gcn, b_gcn, W_out, b_out):
    src = edge_index[0]
    dst = edge_index[1]
    pad = EP - E
    src_p = jnp.concatenate([src, jnp.zeros((pad,), jnp.int32)])
    dst_p = jnp.concatenate([dst, jnp.full((pad,), N + 16, jnp.int32)])
    dst2d = dst_p.reshape(EP // BB, BB)
    x_pad = jnp.pad(x, ((0, NP - N), (0, 0)))

    degp = _degree_partials(dst2d)
    t, dinv = _tc_mid(x_pad, W_proj, b_proj, W_gcn, W_out, degp)
    sp = _message_partials(src_p, dst2d, t)
    out_full = _tc_final(dinv, t, sp, W_out, b_gcn, b_out)
    return out_full[:N, None]


# speedup vs baseline: 146.8983x; 1.4237x over previous
import functools

import jax
import jax.numpy as jnp
from jax import lax
from jax.experimental import pallas as pl
from jax.experimental.pallas import tpu as pltpu
from jax.experimental.pallas import tpu_sc as plsc

N = 10000
E = 320000
NC = 2
NS = 16
L = 16
NW = NC * NS
NP = 10240
BB = 128
ROWS = E // BB
NB = 80
NBL = 20
G = 20
NSL = NP // NS
MROWS = 1024


def _sc_mesh():
    return plsc.VectorSubcoreMesh(core_axis_name="c", subcore_axis_name="s")


_SC_PARAMS = pltpu.CompilerParams(needs_layout_passes=False)


def _zero_acc_slice(zero_v, acc_sh, sid):
    for i in range(NSL // L):
        zero_v[pl.ds(i * L, L)] = jnp.zeros((L,), jnp.float32)
    pltpu.sync_copy(zero_v, acc_sh.at[pl.ds(sid * NSL, NSL)])


def _stage_edge_rows(ei3_hbm, row, buf_v, wid):
    last = wid == NW - 1

    @pl.when(jnp.logical_not(last))
    def _():
        pltpu.sync_copy(ei3_hbm.at[row, pl.ds(wid * NB, NB)], buf_v)

    @pl.when(last)
    def _():
        pltpu.sync_copy(ei3_hbm.at[row, pl.ds(ROWS - NBL, NBL)],
                        buf_v.at[pl.ds(0, NBL)])

    return jnp.where(last, NBL // G, NB // G)


def _deg_body(ei3_hbm, out_hbm, didx_v, ones_v, zero_v, acc_sh, sem):
    cid = lax.axis_index("c")
    sid = lax.axis_index("s")
    wid = cid * NS + sid
    ngroups = _stage_edge_rows(ei3_hbm, 1, didx_v, wid)
    for i in range(BB // L):
        ones_v[pl.ds(i * L, L)] = jnp.ones((L,), jnp.float32)
    _zero_acc_slice(zero_v, acc_sh, sid)
    plsc.subcore_barrier()

    def group(g, carry):
        cps = [
            pltpu.async_copy(ones_v, acc_sh.at[didx_v.at[g * G + jj]],
                             sem, add=True)
            for jj in range(G)
        ]
        for cp in cps:
            cp.wait()
        return carry

    lax.fori_loop(0, ngroups, group, 0)
    plsc.subcore_barrier()
    pltpu.sync_copy(acc_sh.at[pl.ds(sid * NSL, NSL)],
                    out_hbm.at[cid, pl.ds(sid * NSL, NSL)])


def _degree_partials(ei3):
    return pl.kernel(
        _deg_body,
        out_type=jax.ShapeDtypeStruct((NC, NP), jnp.float32),
        mesh=_sc_mesh(),
        compiler_params=_SC_PARAMS,
        scratch_types=[
            pltpu.VMEM((NB, BB), jnp.int32),
            pltpu.VMEM((BB,), jnp.float32),
            pltpu.VMEM((NSL,), jnp.float32),
            pltpu.VMEM_SHARED((NP,), jnp.float32),
            pltpu.SemaphoreType.DMA,
        ],
    )(ei3)


def _msg_body(ei3_hbm, t_hbm, out_hbm,
              sidx_v, didx_v, vals_v, t_v, zero_v, acc_sh, sem):
    cid = lax.axis_index("c")
    sid = lax.axis_index("s")
    wid = cid * NS + sid
    tcp = pltpu.async_copy(t_hbm, t_v, sem)
    ngroups = _stage_edge_rows(ei3_hbm, 0, sidx_v, wid)
    _stage_edge_rows(ei3_hbm, 1, didx_v, wid)
    _zero_acc_slice(zero_v, acc_sh, sid)
    tcp.wait()
    plsc.subcore_barrier()

    def group(g, carry):
        cps = []
        for jj in range(G):
            j = g * G + jj
            for k in range(BB // L):
                si = sidx_v[j, pl.ds(k * L, L)]
                vals_v[j, pl.ds(k * L, L)] = plsc.load_gather(t_v, [si])
            cps.append(
                pltpu.async_copy(vals_v.at[j], acc_sh.at[didx_v.at[j]],
                                 sem, add=True))
        for cp in cps:
            cp.wait()
        return carry

    lax.fori_loop(0, ngroups, group, 0)
    plsc.subcore_barrier()
    pltpu.sync_copy(acc_sh.at[pl.ds(sid * NSL, NSL)],
                    out_hbm.at[cid, pl.ds(sid * NSL, NSL)])


def _message_partials(ei3, t):
    return pl.kernel(
        _msg_body,
        out_type=jax.ShapeDtypeStruct((NC, NP), jnp.float32),
        mesh=_sc_mesh(),
        compiler_params=_SC_PARAMS,
        scratch_types=[
            pltpu.VMEM((NB, BB), jnp.int32),
            pltpu.VMEM((NB, BB), jnp.int32),
            pltpu.VMEM((NB, BB), jnp.float32),
            pltpu.VMEM((NP,), jnp.float32),
            pltpu.VMEM((NSL,), jnp.float32),
            pltpu.VMEM_SHARED((NP,), jnp.float32),
            pltpu.SemaphoreType.DMA,
        ],
    )(ei3, t)


def _mid_body(x_ref, wp_ref, bp_ref, wg_ref, wo_ref, degp_ref,
              t_ref, dinv_ref):
    u = jnp.dot(wo_ref[...][0, :], wg_ref[...])
    h = lax.dot_general(x_ref[...], wp_ref[...],
                        (((1,), (1,)), ((), ())),
                        preferred_element_type=jnp.float32)
    h = jnp.maximum(h + bp_ref[...][None, :], 0.0)
    t_raw = jnp.sum(h * u[None, :], axis=1)
    deg = degp_ref[0, :] + degp_ref[1, :] + 1.0
    dinv = lax.rsqrt(deg)
    t_ref[...] = dinv * t_raw
    dinv_ref[...] = dinv


def _tc_mid(x, W_proj, b_proj, W_gcn, W_out, degp):
    return pl.pallas_call(
        _mid_body,
        grid=(NP // MROWS,),
        in_specs=[
            pl.BlockSpec((MROWS, 128), lambda i: (i, 0)),
            pl.BlockSpec((64, 128), lambda i: (0, 0)),
            pl.BlockSpec((64,), lambda i: (0,)),
            pl.BlockSpec((32, 64), lambda i: (0, 0)),
            pl.BlockSpec((1, 32), lambda i: (0, 0)),
            pl.BlockSpec((NC, MROWS), lambda i: (0, i)),
        ],
        out_specs=[
            pl.BlockSpec((MROWS,), lambda i: (i,)),
            pl.BlockSpec((MROWS,), lambda i: (i,)),
        ],
        out_shape=[
            jax.ShapeDtypeStruct((NP,), jnp.float32),
            jax.ShapeDtypeStruct((NP,), jnp.float32),
        ],
    )(x, W_proj, b_proj, W_gcn, W_out, degp)


def _final_body(dinv_ref, t_ref, sp_ref, wo_ref, bg_ref, bo_ref, out_ref):
    c = jnp.sum(wo_ref[...][0, :] * bg_ref[...]) + jnp.sum(bo_ref[...])
    out_ref[...] = dinv_ref[...] * (sp_ref[0, :] + sp_ref[1, :] + t_ref[...]) + c


def _tc_final(dinv, t, sp, W_out, b_gcn, b_out):
    return pl.pallas_call(
        _final_body,
        out_shape=jax.ShapeDtypeStruct((NP,), jnp.float32),
    )(dinv, t, sp, W_out, b_gcn, b_out)


@jax.jit
def kernel(x, edge_index, W_proj, b_proj, W_gcn, b_gcn, W_out, b_out):
    ei3 = edge_index.reshape(2, ROWS, BB)
    degp = _degree_partials(ei3)
    t, dinv = _tc_mid(x, W_proj, b_proj, W_gcn, W_out, degp)
    sp = _message_partials(ei3, t)
    out_full = _tc_final(dinv, t, sp, W_out, b_gcn, b_out)
    return out_full[:N, None]
